# baseline (device time: 19035 ns/iter reference)
import jax
import jax.numpy as jnp
from jax import lax
from jax.experimental import pallas as pl
from jax.experimental.pallas import tpu as pltpu


def kernel(x, dest):
    m, n = x.shape
    my_y = lax.axis_index("y")

    keep = (dest == my_y).astype(jnp.int32)
    k = jnp.sum(keep)
    p = m - k
    order = jnp.argsort(keep, stable=True)
    x_sorted = x[order]

    def body(xs_ref, recv_ref, send_sem, recv_sem):
        my_x = lax.axis_index("x")
        peer = (my_x, 1 - lax.axis_index("y"))
        barrier = pltpu.get_barrier_semaphore()
        pl.semaphore_signal(
            barrier, inc=1, device_id=peer,
            device_id_type=pl.DeviceIdType.MESH,
        )
        pl.semaphore_wait(barrier, 1)
        rdma = pltpu.make_async_remote_copy(
            src_ref=xs_ref,
            dst_ref=recv_ref,
            send_sem=send_sem,
            recv_sem=recv_sem,
            device_id=peer,
            device_id_type=pl.DeviceIdType.MESH,
        )
        rdma.start()
        rdma.wait()

    recv = pl.pallas_call(
        body,
        out_shape=jax.ShapeDtypeStruct((m, n), x.dtype),
        in_specs=[pl.BlockSpec(memory_space=pltpu.VMEM)],
        out_specs=pl.BlockSpec(memory_space=pltpu.VMEM),
        scratch_shapes=[pltpu.SemaphoreType.DMA, pltpu.SemaphoreType.DMA],
        compiler_params=pltpu.CompilerParams(collective_id=0),
    )(x_sorted)

    i = jnp.arange(m)
    cat = jnp.concatenate([x_sorted, recv], axis=0)
    idx0 = jnp.where(i < k, p + i, m + (i - k))
    idx1 = jnp.where(i < p, m + i, i)
    idx = jnp.where(my_y == 0, idx0, idx1)
    return cat[idx]


# device time: 15023 ns/iter; 1.2671x vs baseline; 1.2671x over previous
import jax
import jax.numpy as jnp
from jax import lax
from jax.experimental import pallas as pl
from jax.experimental.pallas import tpu as pltpu

C = 64
MAX_CHUNKS = 8


def kernel(x, dest):
    m, n = x.shape
    my_y = lax.axis_index("y")

    keep = dest == my_y
    ki = keep.astype(jnp.int32)
    c_keep = jnp.cumsum(ki)
    c_peer = jnp.cumsum(1 - ki)
    k = c_keep[-1]
    p = m - k
    dst_pos = jnp.where(keep, p + c_keep - 1, c_peer - 1)
    order = jnp.zeros((m,), jnp.int32).at[dst_pos].set(
        jnp.arange(m, dtype=jnp.int32)
    )
    x_sorted = x[order]
    k_arr = jnp.reshape(k, (1,)).astype(jnp.int32)

    def body(k_ref, xs_ref, out_ref, recv_ref, send_sems, recv_sems):
        my_x = lax.axis_index("x")
        yy = lax.axis_index("y")
        peer = (my_x, 1 - yy)
        k_ = k_ref[0]
        p_ = m - k_
        n_ch = (p_ + C - 1) // C

        barrier = pltpu.get_barrier_semaphore()
        pl.semaphore_signal(
            barrier, inc=1, device_id=peer,
            device_id_type=pl.DeviceIdType.MESH,
        )
        pl.semaphore_wait(barrier, 1)

        def chunk(c):
            off = pl.multiple_of(c * C, C)
            return pltpu.make_async_remote_copy(
                src_ref=xs_ref.at[pl.ds(off, C)],
                dst_ref=recv_ref.at[pl.ds(off, C)],
                send_sem=send_sems.at[c],
                recv_sem=recv_sems.at[c],
                device_id=peer,
                device_id_type=pl.DeviceIdType.MESH,
            )

        def issue(c, _):
            chunk(c).start()
            return _
        lax.fori_loop(0, n_ch, issue, None)

        def wait_in(c, _):
            chunk(c).wait_recv()
            return _
        lax.fori_loop(0, n_ch, wait_in, None)

        xs = xs_ref[...]
        rv = recv_ref[...]
        row = lax.broadcasted_iota(jnp.int32, (m, n), 0)
        merged0 = jnp.where(
            row < k_, pltpu.roll(xs, k_, 0), pltpu.roll(rv, k_, 0)
        )
        merged1 = jnp.where(row < p_, rv, xs)
        out_ref[...] = jnp.where(yy == 0, merged0, merged1)

        def wait_out(c, _):
            chunk(c).wait_send()
            return _
        lax.fori_loop(0, n_ch, wait_out, None)

    return pl.pallas_call(
        body,
        out_shape=jax.ShapeDtypeStruct((m, n), x.dtype),
        in_specs=[
            pl.BlockSpec(memory_space=pltpu.SMEM),
            pl.BlockSpec(memory_space=pltpu.VMEM),
        ],
        out_specs=pl.BlockSpec(memory_space=pltpu.VMEM),
        scratch_shapes=[
            pltpu.VMEM((m, n), x.dtype),
            pltpu.SemaphoreType.DMA((MAX_CHUNKS,)),
            pltpu.SemaphoreType.DMA((MAX_CHUNKS,)),
        ],
        compiler_params=pltpu.CompilerParams(collective_id=0),
    )(k_arr, x_sorted)


# device time: 13030 ns/iter; 1.4609x vs baseline; 1.1530x over previous
import jax
import jax.numpy as jnp
from jax import lax
from jax.experimental import pallas as pl
from jax.experimental.pallas import tpu as pltpu

C = 64
MAX_CHUNKS = 8


def kernel(x, dest):
    m, n = x.shape
    my_y = lax.axis_index("y")

    keep = dest == my_y
    ki = keep.astype(jnp.int32)
    c_keep = jnp.cumsum(ki)
    c_peer = jnp.arange(1, m + 1, dtype=jnp.int32) - c_keep
    k = c_keep[-1]
    p = m - k
    dst_pos = jnp.where(keep, p + c_keep - 1, c_peer - 1)
    x_sorted = jnp.zeros_like(x).at[dst_pos].set(
        x, unique_indices=True, mode="promise_in_bounds"
    )
    k_arr = jnp.reshape(k, (1,)).astype(jnp.int32)

    def body(k_ref, xs_ref, out_ref, recv_ref, send_sems, recv_sems):
        my_x = lax.axis_index("x")
        yy = lax.axis_index("y")
        peer = (my_x, 1 - yy)
        k_ = k_ref[0]
        p_ = m - k_
        n_ch = (p_ + C - 1) // C

        barrier = pltpu.get_barrier_semaphore()
        pl.semaphore_signal(
            barrier, inc=1, device_id=peer,
            device_id_type=pl.DeviceIdType.MESH,
        )
        pl.semaphore_wait(barrier, 1)

        def chunk(c):
            off = pl.multiple_of(c * C, C)
            return pltpu.make_async_remote_copy(
                src_ref=xs_ref.at[pl.ds(off, C)],
                dst_ref=recv_ref.at[pl.ds(off, C)],
                send_sem=send_sems.at[c],
                recv_sem=recv_sems.at[c],
                device_id=peer,
                device_id_type=pl.DeviceIdType.MESH,
            )

        def issue(c, _):
            chunk(c).start()
            return _
        lax.fori_loop(0, n_ch, issue, None)

        def wait_in(c, _):
            chunk(c).wait_recv()
            return _
        lax.fori_loop(0, n_ch, wait_in, None)

        xs = xs_ref[...]
        rv = recv_ref[...]
        row = lax.broadcasted_iota(jnp.int32, (m, n), 0)
        z = jnp.where(row < p_, rv, xs)

        @pl.when(yy == 0)
        def _():
            out_ref[...] = pltpu.roll(z, k_, 0)

        @pl.when(yy != 0)
        def _():
            out_ref[...] = z

        def wait_out(c, _):
            chunk(c).wait_send()
            return _
        lax.fori_loop(0, n_ch, wait_out, None)

    return pl.pallas_call(
        body,
        out_shape=jax.ShapeDtypeStruct((m, n), x.dtype),
        in_specs=[
            pl.BlockSpec(memory_space=pltpu.SMEM),
            pl.BlockSpec(memory_space=pltpu.VMEM),
        ],
        out_specs=pl.BlockSpec(memory_space=pltpu.VMEM),
        scratch_shapes=[
            pltpu.VMEM((m, n), x.dtype),
            pltpu.SemaphoreType.DMA((MAX_CHUNKS,)),
            pltpu.SemaphoreType.DMA((MAX_CHUNKS,)),
        ],
        compiler_params=pltpu.CompilerParams(collective_id=0),
    )(k_arr, x_sorted)


# device time: 11523 ns/iter; 1.6519x vs baseline; 1.1308x over previous
import jax
import jax.numpy as jnp
from jax import lax
from jax.experimental import pallas as pl
from jax.experimental.pallas import tpu as pltpu

C = 64
MAX_CHUNKS = 8


def kernel(x, dest):
    m, n = x.shape
    my_y = lax.axis_index("y")

    keep = dest == my_y
    ki = keep.astype(jnp.int32)
    c_keep = jnp.cumsum(ki)
    c_peer = jnp.arange(1, m + 1, dtype=jnp.int32) - c_keep
    k = c_keep[-1]
    p = m - k
    dst_pos = jnp.where(keep, p + c_keep - 1, c_peer - 1)
    dp_2d = jnp.reshape(dst_pos, (1, m))
    k_arr = jnp.reshape(k, (1,)).astype(jnp.int32)

    def body(k_ref, dp_ref, x_ref, out_ref, xs_ref, recv_ref,
             send_sems, recv_sems):
        my_x = lax.axis_index("x")
        yy = lax.axis_index("y")
        peer = (my_x, 1 - yy)
        k_ = k_ref[0]
        p_ = m - k_
        n_ch = (p_ + C - 1) // C

        barrier = pltpu.get_barrier_semaphore()
        pl.semaphore_signal(
            barrier, inc=1, device_id=peer,
            device_id_type=pl.DeviceIdType.MESH,
        )

        row_i = lax.broadcasted_iota(jnp.int32, (m, m), 0)
        perm = (row_i == dp_ref[...]).astype(x_ref.dtype)
        xs = jnp.dot(
            perm, x_ref[...], preferred_element_type=jnp.float32
        )
        xs_ref[...] = xs

        pl.semaphore_wait(barrier, 1)

        def chunk(c):
            off = pl.multiple_of(c * C, C)
            return pltpu.make_async_remote_copy(
                src_ref=xs_ref.at[pl.ds(off, C)],
                dst_ref=recv_ref.at[pl.ds(off, C)],
                send_sem=send_sems.at[c],
                recv_sem=recv_sems.at[c],
                device_id=peer,
                device_id_type=pl.DeviceIdType.MESH,
            )

        def issue(c, _):
            chunk(c).start()
            return _
        lax.fori_loop(0, n_ch, issue, None)

        def wait_in(c, _):
            chunk(c).wait_recv()
            return _
        lax.fori_loop(0, n_ch, wait_in, None)

        rv = recv_ref[...]
        row = lax.broadcasted_iota(jnp.int32, (m, n), 0)
        z = jnp.where(row < p_, rv, xs)

        @pl.when(yy == 0)
        def _():
            out_ref[...] = pltpu.roll(z, k_, 0)

        @pl.when(yy != 0)
        def _():
            out_ref[...] = z

        def wait_out(c, _):
            chunk(c).wait_send()
            return _
        lax.fori_loop(0, n_ch, wait_out, None)

    return pl.pallas_call(
        body,
        out_shape=jax.ShapeDtypeStruct((m, n), x.dtype),
        in_specs=[
            pl.BlockSpec(memory_space=pltpu.SMEM),
            pl.BlockSpec(memory_space=pltpu.VMEM),
            pl.BlockSpec(memory_space=pltpu.VMEM),
        ],
        out_specs=pl.BlockSpec(memory_space=pltpu.VMEM),
        scratch_shapes=[
            pltpu.VMEM((m, n), x.dtype),
            pltpu.VMEM((m, n), x.dtype),
            pltpu.SemaphoreType.DMA((MAX_CHUNKS,)),
            pltpu.SemaphoreType.DMA((MAX_CHUNKS,)),
        ],
        compiler_params=pltpu.CompilerParams(collective_id=0),
    )(k_arr, dp_2d, x)
